# baseline (speedup 1.0000x reference)
import jax
import jax.numpy as jnp
from jax.experimental import pallas as pl
from jax.experimental.pallas import tpu as pltpu
from jax.experimental.pallas import tpu_sc as plsc

N = 10000
E = 320000
D_NODE = 128
D_EATTR = 16
D_EH = 64
D_H = 128

_CHUNKS = 4
_EC = E // _CHUNKS
_BLK = 1600
_CBLOCKS = _EC // _BLK
_GW = 256


def _silu(x):
    return x * jax.nn.sigmoid(x)


def _bdot(a, b):
    return jnp.dot(a.astype(jnp.bfloat16), b.astype(jnp.bfloat16),
                   preferred_element_type=jnp.float32)


def _ln(x, g, b):
    m = jnp.mean(x, axis=-1, keepdims=True)
    v = jnp.mean((x - m) ** 2, axis=-1, keepdims=True)
    return (x - m) * jax.lax.rsqrt(v + 1e-5) * g + b


def _table_body(ne_ref, lg_ref, wsrc_ref, wdst_ref, was_ref, wad_ref, out_ref):
    ne = ne_ref[...]
    ap = jax.nn.sigmoid(lg_ref[...])
    out_ref[0] = jnp.dot(ne, wsrc_ref[...]) + ap * was_ref[...]
    out_ref[1] = jnp.dot(ne, wdst_ref[...]) + ap * wad_ref[...]


def _build_table(node_embedding, logits, wsrc, wdst, w_as, w_ad):
    out = pl.pallas_call(
        _table_body,
        out_shape=jax.ShapeDtypeStruct((2, N, D_NODE), jnp.float32),
    )(node_embedding, logits.reshape(N, 1), wsrc, wdst, w_as, w_ad)
    return out.reshape(2 * N, D_NODE)


def _sc_gather(table, indices):
    n_idx = indices.shape[0]
    idx2 = indices.reshape(1, n_idx)
    mesh = plsc.VectorSubcoreMesh(core_axis_name="core", subcore_axis_name="subcore")

    @pl.kernel(
        out_type=jax.ShapeDtypeStruct((n_idx, D_NODE), jnp.float32),
        mesh=mesh,
    )
    def gather_kernel(tbl_hbm, i_hbm, o_hbm):
        def body(i_vmem, o_vmem):
            pltpu.sync_copy(tbl_hbm.at[i_vmem.at[0]], o_vmem)

        pltpu.emit_pipeline(
            body,
            grid=(n_idx // _GW,),
            in_specs=[pl.BlockSpec((1, _GW), lambda i: (0, i))],
            out_specs=[pl.BlockSpec((_GW, D_NODE), lambda i: (i, 0))],
            core_axis_name=("core", "subcore"),
            dimension_semantics=(pltpu.PARALLEL,),
        )(i_hbm, o_hbm)

    return gather_kernel(table, idx2)


def _edge_body(ea_ref, dn_ref, g_ref,
               be1w_ref, be1b_ref, be1g_ref, be1be_ref,
               be2w_ref, be2b_ref, be2g_ref, be2be_ref,
               be3w_ref, be3b_ref, be3g_ref, be3be_ref,
               dg1w_ref, dg1b_ref, dg2w_ref, dg2b_ref, dgow_ref, dgob_ref,
               we_ref, wd_ref, fc1b_ref,
               fc2w_ref, fc2b_ref, outw_ref, outb_ref,
               out_ref):
    ea = ea_ref[...].astype(jnp.bfloat16)
    e = _silu(_ln(_bdot(ea, be1w_ref[...]) + be1b_ref[...],
                  be1g_ref[...], be1be_ref[...]))
    e = _silu(_ln(_bdot(e, be2w_ref[...]) + be2b_ref[...],
                  be2g_ref[...], be2be_ref[...]))
    e = _silu(_ln(_bdot(e, be3w_ref[...]) + be3b_ref[...],
                  be3g_ref[...], be3be_ref[...]))

    dn = dn_ref[...].astype(jnp.bfloat16)
    h = _silu(_bdot(dn, dg1w_ref[...]) + dg1b_ref[...])
    h = _silu(_bdot(h, dg2w_ref[...]) + dg2b_ref[...])
    dlog = jnp.sum(h * dgow_ref[...], axis=-1, keepdims=True) + dgob_ref[...]
    dp = jax.nn.sigmoid(dlog)

    x1 = _silu(g_ref[0] + g_ref[1]
               + _bdot(e, we_ref[...])
               + dp * wd_ref[...] + fc1b_ref[...])
    x2 = _silu(_bdot(x1, fc2w_ref[...]) + fc2b_ref[...])
    out_ref[...] = jnp.sum(x2 * outw_ref[...], axis=-1, keepdims=True) + outb_ref[0, 0]


def _full(arr):
    nd = arr.ndim
    return pl.BlockSpec(arr.shape, lambda i, _n=nd: (0,) * _n)


def kernel(node_embedding, edge_index, edge_attr, dual_node_emb, atom_reactivity_logits,
           be1_W, be1_b, be1_g, be1_beta,
           be2_W, be2_b, be2_g, be2_beta,
           be3_W, be3_b, be3_g, be3_beta,
           dg1_W, dg1_b, dg2_W, dg2_b, dgo_W, dgo_b,
           fc1_W, fc1_b, fc2_W, fc2_b, out_W, out_b):
    src = edge_index[0].astype(jnp.int32)
    dst = edge_index[1].astype(jnp.int32)

    wsrc = fc1_W[:D_NODE]
    wdst = fc1_W[D_NODE:2 * D_NODE]
    w_e = fc1_W[2 * D_NODE:2 * D_NODE + D_EH]
    w_d = fc1_W[2 * D_NODE + D_EH:2 * D_NODE + D_EH + 1]
    w_as = fc1_W[2 * D_NODE + D_EH + 1:2 * D_NODE + D_EH + 2]
    w_ad = fc1_W[2 * D_NODE + D_EH + 2:2 * D_NODE + D_EH + 3]

    table = _build_table(node_embedding, atom_reactivity_logits, wsrc, wdst, w_as, w_ad)

    weights = [be1_W, be1_b.reshape(1, -1), be1_g.reshape(1, -1), be1_beta.reshape(1, -1),
               be2_W, be2_b.reshape(1, -1), be2_g.reshape(1, -1), be2_beta.reshape(1, -1),
               be3_W, be3_b.reshape(1, -1), be3_g.reshape(1, -1), be3_beta.reshape(1, -1),
               dg1_W, dg1_b.reshape(1, -1), dg2_W, dg2_b.reshape(1, -1),
               dgo_W.reshape(1, -1), dgo_b.reshape(1, 1),
               w_e, w_d, fc1_b.reshape(1, -1),
               fc2_W, fc2_b.reshape(1, -1), out_W.reshape(1, -1), out_b.reshape(1, 1)]

    gathers = []
    for c in range(_CHUNKS):
        s = c * _EC
        idx_c = jnp.concatenate([src[s:s + _EC], dst[s:s + _EC] + N])
        gathers.append(_sc_gather(table, idx_c))

    outs = []
    for c in range(_CHUNKS):
        in_specs = [
            pl.BlockSpec((_BLK, D_EATTR), lambda i, _c=c: (i + _c * _CBLOCKS, 0)),
            pl.BlockSpec((_BLK, D_EH), lambda i, _c=c: (i + _c * _CBLOCKS, 0)),
            pl.BlockSpec((2, _BLK, D_NODE), lambda i: (0, i, 0)),
        ] + [_full(w) for w in weights]

        out_c = pl.pallas_call(
            _edge_body,
            grid=(_CBLOCKS,),
            in_specs=in_specs,
            out_specs=pl.BlockSpec((_BLK, 1), lambda i: (i, 0)),
            out_shape=jax.ShapeDtypeStruct((_EC, 1), jnp.float32),
        )(edge_attr, dual_node_emb,
          gathers[c].reshape(2, _EC, D_NODE), *weights)
        outs.append(out_c)
    return jnp.concatenate(outs, axis=0).reshape(E)


# speedup vs baseline: 1.0001x; 1.0001x over previous
import jax
import jax.numpy as jnp
from jax.experimental import pallas as pl
from jax.experimental.pallas import tpu as pltpu
from jax.experimental.pallas import tpu_sc as plsc

N = 10000
E = 320000
D_NODE = 128
D_EATTR = 16
D_EH = 64
D_H = 128

_CHUNKS = 8
_EC = E // _CHUNKS
_BLK = 1600
_CBLOCKS = _EC // _BLK
_GW = 256


def _silu(x):
    return x * jax.nn.sigmoid(x)


def _bdot(a, b):
    return jnp.dot(a.astype(jnp.bfloat16), b.astype(jnp.bfloat16),
                   preferred_element_type=jnp.float32)


def _ln(x, g, b):
    m = jnp.mean(x, axis=-1, keepdims=True)
    v = jnp.mean((x - m) ** 2, axis=-1, keepdims=True)
    return (x - m) * jax.lax.rsqrt(v + 1e-5) * g + b


def _table_body(ne_ref, lg_ref, wsrc_ref, wdst_ref, was_ref, wad_ref, out_ref):
    ne = ne_ref[...]
    ap = jax.nn.sigmoid(lg_ref[...])
    p = jnp.dot(ne, wsrc_ref[...]) + ap * was_ref[...]
    q = jnp.dot(ne, wdst_ref[...]) + ap * wad_ref[...]
    out_ref[0] = p
    out_ref[1] = q


def _build_table(node_embedding, logits, wsrc, wdst, w_as, w_ad):
    out = pl.pallas_call(
        _table_body,
        out_shape=jax.ShapeDtypeStruct((2, N, D_NODE), jnp.float32),
    )(node_embedding, logits.reshape(N, 1), wsrc, wdst, w_as, w_ad)
    return out.reshape(2 * N, D_NODE)


def _sc_gather(table, indices):
    n_idx = indices.shape[0]
    idx2 = indices.reshape(1, n_idx)
    mesh = plsc.VectorSubcoreMesh(core_axis_name="core", subcore_axis_name="subcore")

    @pl.kernel(
        out_type=jax.ShapeDtypeStruct((n_idx, D_NODE), jnp.float32),
        mesh=mesh,
    )
    def gather_kernel(tbl_hbm, i_hbm, o_hbm):
        def body(i_vmem, o_vmem):
            pltpu.sync_copy(tbl_hbm.at[i_vmem.at[0]], o_vmem)

        pltpu.emit_pipeline(
            body,
            grid=(n_idx // _GW,),
            in_specs=[pl.BlockSpec((1, _GW), lambda i: (0, i))],
            out_specs=[pl.BlockSpec((_GW, D_NODE), lambda i: (i, 0))],
            core_axis_name=("core", "subcore"),
            dimension_semantics=(pltpu.PARALLEL,),
        )(i_hbm, o_hbm)

    return gather_kernel(table, idx2)


def _edge_body(ea_ref, dn_ref, g_ref,
               be1w_ref, be1b_ref, be1g_ref, be1be_ref,
               be2w_ref, be2b_ref, be2g_ref, be2be_ref,
               be3w_ref, be3b_ref, be3g_ref, be3be_ref,
               dg1w_ref, dg1b_ref, dg2w_ref, dg2b_ref, dgow_ref, dgob_ref,
               we_ref, wd_ref, fc1b_ref,
               fc2w_ref, fc2b_ref, outw_ref, outb_ref,
               out_ref):
    ea = ea_ref[...].astype(jnp.bfloat16)
    e = _silu(_ln(_bdot(ea, be1w_ref[...]) + be1b_ref[...],
                  be1g_ref[...], be1be_ref[...]))
    e = _silu(_ln(_bdot(e, be2w_ref[...]) + be2b_ref[...],
                  be2g_ref[...], be2be_ref[...]))
    e = _silu(_ln(_bdot(e, be3w_ref[...]) + be3b_ref[...],
                  be3g_ref[...], be3be_ref[...]))

    dn = dn_ref[...].astype(jnp.bfloat16)
    h = _silu(_bdot(dn, dg1w_ref[...]) + dg1b_ref[...])
    h = _silu(_bdot(h, dg2w_ref[...]) + dg2b_ref[...])
    dlog = jnp.sum(h * dgow_ref[...], axis=-1, keepdims=True) + dgob_ref[...]
    dp = jax.nn.sigmoid(dlog)

    x1 = _silu(g_ref[0] + g_ref[1]
               + _bdot(e, we_ref[...])
               + dp * wd_ref[...] + fc1b_ref[...])
    x2 = _silu(_bdot(x1, fc2w_ref[...]) + fc2b_ref[...])
    out_ref[...] = jnp.sum(x2 * outw_ref[...], axis=-1, keepdims=True) + outb_ref[0, 0]


def _full(arr):
    nd = arr.ndim
    return pl.BlockSpec(arr.shape, lambda i, _n=nd: (0,) * _n)


def kernel(node_embedding, edge_index, edge_attr, dual_node_emb, atom_reactivity_logits,
           be1_W, be1_b, be1_g, be1_beta,
           be2_W, be2_b, be2_g, be2_beta,
           be3_W, be3_b, be3_g, be3_beta,
           dg1_W, dg1_b, dg2_W, dg2_b, dgo_W, dgo_b,
           fc1_W, fc1_b, fc2_W, fc2_b, out_W, out_b):
    src = edge_index[0].astype(jnp.int32)
    dst = edge_index[1].astype(jnp.int32)

    wsrc = fc1_W[:D_NODE]
    wdst = fc1_W[D_NODE:2 * D_NODE]
    w_e = fc1_W[2 * D_NODE:2 * D_NODE + D_EH]
    w_d = fc1_W[2 * D_NODE + D_EH:2 * D_NODE + D_EH + 1]
    w_as = fc1_W[2 * D_NODE + D_EH + 1:2 * D_NODE + D_EH + 2]
    w_ad = fc1_W[2 * D_NODE + D_EH + 2:2 * D_NODE + D_EH + 3]

    table = _build_table(node_embedding, atom_reactivity_logits, wsrc, wdst, w_as, w_ad)

    weights = [be1_W, be1_b.reshape(1, -1), be1_g.reshape(1, -1), be1_beta.reshape(1, -1),
               be2_W, be2_b.reshape(1, -1), be2_g.reshape(1, -1), be2_beta.reshape(1, -1),
               be3_W, be3_b.reshape(1, -1), be3_g.reshape(1, -1), be3_beta.reshape(1, -1),
               dg1_W, dg1_b.reshape(1, -1), dg2_W, dg2_b.reshape(1, -1),
               dgo_W.reshape(1, -1), dgo_b.reshape(1, 1),
               w_e, w_d, fc1_b.reshape(1, -1),
               fc2_W, fc2_b.reshape(1, -1), out_W.reshape(1, -1), out_b.reshape(1, 1)]

    gathers = []
    for c in range(_CHUNKS):
        s = c * _EC
        idx_c = jnp.concatenate([src[s:s + _EC], dst[s:s + _EC] + N])
        gathers.append(_sc_gather(table, idx_c))

    outs = []
    for c in range(_CHUNKS):
        in_specs = [
            pl.BlockSpec((_BLK, D_EATTR), lambda i, _c=c: (i + _c * _CBLOCKS, 0)),
            pl.BlockSpec((_BLK, D_EH), lambda i, _c=c: (i + _c * _CBLOCKS, 0)),
            pl.BlockSpec((2, _BLK, D_NODE), lambda i: (0, i, 0)),
        ] + [_full(w) for w in weights]

        out_c = pl.pallas_call(
            _edge_body,
            grid=(_CBLOCKS,),
            in_specs=in_specs,
            out_specs=pl.BlockSpec((_BLK, 1), lambda i: (i, 0)),
            out_shape=jax.ShapeDtypeStruct((_EC, 1), jnp.float32),
        )(edge_attr, dual_node_emb,
          gathers[c].reshape(2, _EC, D_NODE), *weights)
        outs.append(out_c)
    return jnp.concatenate(outs, axis=0).reshape(E)


# speedup vs baseline: 1.0377x; 1.0375x over previous
import jax
import jax.numpy as jnp
from jax.experimental import pallas as pl
from jax.experimental.pallas import tpu as pltpu
from jax.experimental.pallas import tpu_sc as plsc

N = 10000
E = 320000
D_NODE = 128
D_EATTR = 16
D_EH = 64
D_H = 128

_CHUNKS = 8
_EC = E // _CHUNKS
_BLK = 1600
_CBLOCKS = _EC // _BLK
_GW = 256


def _silu(x):
    return 0.5 * x * (jnp.tanh(0.5 * x) + 1.0)


def _sigmoid(x):
    return 0.5 * (jnp.tanh(0.5 * x) + 1.0)


def _bdot(a, b):
    return jnp.dot(a.astype(jnp.bfloat16), b.astype(jnp.bfloat16),
                   preferred_element_type=jnp.float32)


def _ln(x, g, b):
    m = jnp.mean(x, axis=-1, keepdims=True)
    v = jnp.mean((x - m) ** 2, axis=-1, keepdims=True)
    return (x - m) * jax.lax.rsqrt(v + 1e-5) * g + b


def _table_body(ne_ref, lg_ref, wsrc_ref, wdst_ref, was_ref, wad_ref, out_ref):
    ne = ne_ref[...]
    ap = jax.nn.sigmoid(lg_ref[...])
    p = jnp.dot(ne, wsrc_ref[...]) + ap * was_ref[...]
    q = jnp.dot(ne, wdst_ref[...]) + ap * wad_ref[...]
    out_ref[0] = p
    out_ref[1] = q


def _build_table(node_embedding, logits, wsrc, wdst, w_as, w_ad):
    out = pl.pallas_call(
        _table_body,
        out_shape=jax.ShapeDtypeStruct((2, N, D_NODE), jnp.float32),
    )(node_embedding, logits.reshape(N, 1), wsrc, wdst, w_as, w_ad)
    return out.reshape(2 * N, D_NODE)


def _sc_gather(table, indices):
    n_idx = indices.shape[0]
    idx2 = indices.reshape(1, n_idx)
    mesh = plsc.VectorSubcoreMesh(core_axis_name="core", subcore_axis_name="subcore")

    @pl.kernel(
        out_type=jax.ShapeDtypeStruct((n_idx, D_NODE), jnp.float32),
        mesh=mesh,
    )
    def gather_kernel(tbl_hbm, i_hbm, o_hbm):
        def body(i_vmem, o_vmem):
            pltpu.sync_copy(tbl_hbm.at[i_vmem.at[0]], o_vmem)

        pltpu.emit_pipeline(
            body,
            grid=(n_idx // _GW,),
            in_specs=[pl.BlockSpec((1, _GW), lambda i: (0, i))],
            out_specs=[pl.BlockSpec((_GW, D_NODE), lambda i: (i, 0))],
            core_axis_name=("core", "subcore"),
            dimension_semantics=(pltpu.PARALLEL,),
        )(i_hbm, o_hbm)

    return gather_kernel(table, idx2)


def _edge_body(ea_ref, dn_ref, g_ref,
               be1w_ref, be1b_ref, be1g_ref, be1be_ref,
               be2w_ref, be2b_ref, be2g_ref, be2be_ref,
               be3w_ref, be3b_ref, be3g_ref, be3be_ref,
               dg1w_ref, dg1b_ref, dg2w_ref, dg2b_ref, dgow_ref, dgob_ref,
               we_ref, wd_ref, fc1b_ref,
               fc2w_ref, fc2b_ref, outw_ref, outb_ref,
               out_ref):
    ea = ea_ref[...].astype(jnp.bfloat16)
    e = _silu(_ln(_bdot(ea, be1w_ref[...]) + be1b_ref[...],
                  be1g_ref[...], be1be_ref[...]))
    e = _silu(_ln(_bdot(e, be2w_ref[...]) + be2b_ref[...],
                  be2g_ref[...], be2be_ref[...]))
    e = _silu(_ln(_bdot(e, be3w_ref[...]) + be3b_ref[...],
                  be3g_ref[...], be3be_ref[...]))

    dn = dn_ref[...].astype(jnp.bfloat16)
    h = _silu(_bdot(dn, dg1w_ref[...]) + dg1b_ref[...])
    h = _silu(_bdot(h, dg2w_ref[...]) + dg2b_ref[...])
    dlog = jnp.sum(h * dgow_ref[...], axis=-1, keepdims=True) + dgob_ref[...]
    dp = _sigmoid(dlog)

    x1 = _silu(g_ref[0] + g_ref[1]
               + _bdot(e, we_ref[...])
               + dp * wd_ref[...] + fc1b_ref[...])
    x2 = _silu(_bdot(x1, fc2w_ref[...]) + fc2b_ref[...])
    out_ref[...] = jnp.sum(x2 * outw_ref[...], axis=-1, keepdims=True) + outb_ref[0, 0]


def _full(arr):
    nd = arr.ndim
    return pl.BlockSpec(arr.shape, lambda i, _n=nd: (0,) * _n)


def kernel(node_embedding, edge_index, edge_attr, dual_node_emb, atom_reactivity_logits,
           be1_W, be1_b, be1_g, be1_beta,
           be2_W, be2_b, be2_g, be2_beta,
           be3_W, be3_b, be3_g, be3_beta,
           dg1_W, dg1_b, dg2_W, dg2_b, dgo_W, dgo_b,
           fc1_W, fc1_b, fc2_W, fc2_b, out_W, out_b):
    src = edge_index[0].astype(jnp.int32)
    dst = edge_index[1].astype(jnp.int32)

    wsrc = fc1_W[:D_NODE]
    wdst = fc1_W[D_NODE:2 * D_NODE]
    w_e = fc1_W[2 * D_NODE:2 * D_NODE + D_EH]
    w_d = fc1_W[2 * D_NODE + D_EH:2 * D_NODE + D_EH + 1]
    w_as = fc1_W[2 * D_NODE + D_EH + 1:2 * D_NODE + D_EH + 2]
    w_ad = fc1_W[2 * D_NODE + D_EH + 2:2 * D_NODE + D_EH + 3]

    table = _build_table(node_embedding, atom_reactivity_logits, wsrc, wdst, w_as, w_ad)

    weights = [be1_W, be1_b.reshape(1, -1), be1_g.reshape(1, -1), be1_beta.reshape(1, -1),
               be2_W, be2_b.reshape(1, -1), be2_g.reshape(1, -1), be2_beta.reshape(1, -1),
               be3_W, be3_b.reshape(1, -1), be3_g.reshape(1, -1), be3_beta.reshape(1, -1),
               dg1_W, dg1_b.reshape(1, -1), dg2_W, dg2_b.reshape(1, -1),
               dgo_W.reshape(1, -1), dgo_b.reshape(1, 1),
               w_e, w_d, fc1_b.reshape(1, -1),
               fc2_W, fc2_b.reshape(1, -1), out_W.reshape(1, -1), out_b.reshape(1, 1)]

    gathers = []
    for c in range(_CHUNKS):
        s = c * _EC
        idx_c = jnp.concatenate([src[s:s + _EC], dst[s:s + _EC] + N])
        gathers.append(_sc_gather(table, idx_c))

    outs = []
    for c in range(_CHUNKS):
        in_specs = [
            pl.BlockSpec((_BLK, D_EATTR), lambda i, _c=c: (i + _c * _CBLOCKS, 0)),
            pl.BlockSpec((_BLK, D_EH), lambda i, _c=c: (i + _c * _CBLOCKS, 0)),
            pl.BlockSpec((2, _BLK, D_NODE), lambda i: (0, i, 0)),
        ] + [_full(w) for w in weights]

        out_c = pl.pallas_call(
            _edge_body,
            grid=(_CBLOCKS,),
            in_specs=in_specs,
            out_specs=pl.BlockSpec((_BLK, 1), lambda i: (i, 0)),
            out_shape=jax.ShapeDtypeStruct((_EC, 1), jnp.float32),
        )(edge_attr, dual_node_emb,
          gathers[c].reshape(2, _EC, D_NODE), *weights)
        outs.append(out_c)
    return jnp.concatenate(outs, axis=0).reshape(E)


# speedup vs baseline: 1.1173x; 1.0767x over previous
import jax
import jax.numpy as jnp
from jax.experimental import pallas as pl
from jax.experimental.pallas import tpu as pltpu
from jax.experimental.pallas import tpu_sc as plsc

N = 10000
E = 320000
D_NODE = 128
D_EATTR = 16
D_EH = 64
D_H = 128

_CHUNKS = 1
_EC = E // _CHUNKS
_BLK = 1600
_CBLOCKS = _EC // _BLK
_GW = 256


def _silu(x):
    return 0.5 * x * (jnp.tanh(0.5 * x) + 1.0)


def _sigmoid(x):
    return 0.5 * (jnp.tanh(0.5 * x) + 1.0)


def _bdot(a, b):
    return jnp.dot(a.astype(jnp.bfloat16), b.astype(jnp.bfloat16),
                   preferred_element_type=jnp.float32)


def _ln(x, g, b):
    m = jnp.mean(x, axis=-1, keepdims=True)
    v = jnp.mean((x - m) ** 2, axis=-1, keepdims=True)
    return (x - m) * jax.lax.rsqrt(v + 1e-5) * g + b


def _table_body(ne_ref, lg_ref, wsrc_ref, wdst_ref, was_ref, wad_ref, out_ref):
    ne = ne_ref[...]
    ap = jax.nn.sigmoid(lg_ref[...])
    p = jnp.dot(ne, wsrc_ref[...]) + ap * was_ref[...]
    q = jnp.dot(ne, wdst_ref[...]) + ap * wad_ref[...]
    out_ref[0] = p
    out_ref[1] = q


def _build_table(node_embedding, logits, wsrc, wdst, w_as, w_ad):
    out = pl.pallas_call(
        _table_body,
        out_shape=jax.ShapeDtypeStruct((2, N, D_NODE), jnp.float32),
    )(node_embedding, logits.reshape(N, 1), wsrc, wdst, w_as, w_ad)
    return out.reshape(2 * N, D_NODE)


def _sc_gather(table, indices):
    n_idx = indices.shape[0]
    idx2 = indices.reshape(1, n_idx)
    mesh = plsc.VectorSubcoreMesh(core_axis_name="core", subcore_axis_name="subcore")

    @pl.kernel(
        out_type=jax.ShapeDtypeStruct((n_idx, D_NODE), jnp.float32),
        mesh=mesh,
    )
    def gather_kernel(tbl_hbm, i_hbm, o_hbm):
        def body(i_vmem, o_vmem):
            pltpu.sync_copy(tbl_hbm.at[i_vmem.at[0]], o_vmem)

        pltpu.emit_pipeline(
            body,
            grid=(n_idx // _GW,),
            in_specs=[pl.BlockSpec((1, _GW), lambda i: (0, i))],
            out_specs=[pl.BlockSpec((_GW, D_NODE), lambda i: (i, 0))],
            core_axis_name=("core", "subcore"),
            dimension_semantics=(pltpu.PARALLEL,),
        )(i_hbm, o_hbm)

    return gather_kernel(table, idx2)


def _edge_body(ea_ref, dn_ref, g_ref,
               be1w_ref, be1b_ref, be1g_ref, be1be_ref,
               be2w_ref, be2b_ref, be2g_ref, be2be_ref,
               be3w_ref, be3b_ref, be3g_ref, be3be_ref,
               dg1w_ref, dg1b_ref, dg2w_ref, dg2b_ref, dgow_ref, dgob_ref,
               we_ref, wd_ref, fc1b_ref,
               fc2w_ref, fc2b_ref, outw_ref, outb_ref,
               out_ref):
    ea = ea_ref[...].astype(jnp.bfloat16)
    e = _silu(_ln(_bdot(ea, be1w_ref[...]) + be1b_ref[...],
                  be1g_ref[...], be1be_ref[...]))
    e = _silu(_ln(_bdot(e, be2w_ref[...]) + be2b_ref[...],
                  be2g_ref[...], be2be_ref[...]))
    e = _silu(_ln(_bdot(e, be3w_ref[...]) + be3b_ref[...],
                  be3g_ref[...], be3be_ref[...]))

    dn = dn_ref[...].astype(jnp.bfloat16)
    h = _silu(_bdot(dn, dg1w_ref[...]) + dg1b_ref[...])
    h = _silu(_bdot(h, dg2w_ref[...]) + dg2b_ref[...])
    dlog = jnp.sum(h * dgow_ref[...], axis=-1, keepdims=True) + dgob_ref[...]
    dp = _sigmoid(dlog)

    x1 = _silu(g_ref[0] + g_ref[1]
               + _bdot(e, we_ref[...])
               + dp * wd_ref[...] + fc1b_ref[...])
    x2 = _silu(_bdot(x1, fc2w_ref[...]) + fc2b_ref[...])
    out_ref[...] = jnp.sum(x2 * outw_ref[...], axis=-1, keepdims=True) + outb_ref[0, 0]


def _full(arr):
    nd = arr.ndim
    return pl.BlockSpec(arr.shape, lambda i, _n=nd: (0,) * _n)


def kernel(node_embedding, edge_index, edge_attr, dual_node_emb, atom_reactivity_logits,
           be1_W, be1_b, be1_g, be1_beta,
           be2_W, be2_b, be2_g, be2_beta,
           be3_W, be3_b, be3_g, be3_beta,
           dg1_W, dg1_b, dg2_W, dg2_b, dgo_W, dgo_b,
           fc1_W, fc1_b, fc2_W, fc2_b, out_W, out_b):
    src = edge_index[0].astype(jnp.int32)
    dst = edge_index[1].astype(jnp.int32)

    wsrc = fc1_W[:D_NODE]
    wdst = fc1_W[D_NODE:2 * D_NODE]
    w_e = fc1_W[2 * D_NODE:2 * D_NODE + D_EH]
    w_d = fc1_W[2 * D_NODE + D_EH:2 * D_NODE + D_EH + 1]
    w_as = fc1_W[2 * D_NODE + D_EH + 1:2 * D_NODE + D_EH + 2]
    w_ad = fc1_W[2 * D_NODE + D_EH + 2:2 * D_NODE + D_EH + 3]

    table = _build_table(node_embedding, atom_reactivity_logits, wsrc, wdst, w_as, w_ad)

    weights = [be1_W, be1_b.reshape(1, -1), be1_g.reshape(1, -1), be1_beta.reshape(1, -1),
               be2_W, be2_b.reshape(1, -1), be2_g.reshape(1, -1), be2_beta.reshape(1, -1),
               be3_W, be3_b.reshape(1, -1), be3_g.reshape(1, -1), be3_beta.reshape(1, -1),
               dg1_W, dg1_b.reshape(1, -1), dg2_W, dg2_b.reshape(1, -1),
               dgo_W.reshape(1, -1), dgo_b.reshape(1, 1),
               w_e, w_d, fc1_b.reshape(1, -1),
               fc2_W, fc2_b.reshape(1, -1), out_W.reshape(1, -1), out_b.reshape(1, 1)]

    gathers = []
    for c in range(_CHUNKS):
        s = c * _EC
        idx_c = jnp.concatenate([src[s:s + _EC], dst[s:s + _EC] + N])
        gathers.append(_sc_gather(table, idx_c))

    outs = []
    for c in range(_CHUNKS):
        in_specs = [
            pl.BlockSpec((_BLK, D_EATTR), lambda i, _c=c: (i + _c * _CBLOCKS, 0)),
            pl.BlockSpec((_BLK, D_EH), lambda i, _c=c: (i + _c * _CBLOCKS, 0)),
            pl.BlockSpec((2, _BLK, D_NODE), lambda i: (0, i, 0)),
        ] + [_full(w) for w in weights]

        out_c = pl.pallas_call(
            _edge_body,
            grid=(_CBLOCKS,),
            in_specs=in_specs,
            out_specs=pl.BlockSpec((_BLK, 1), lambda i: (i, 0)),
            out_shape=jax.ShapeDtypeStruct((_EC, 1), jnp.float32),
        )(edge_attr, dual_node_emb,
          gathers[c].reshape(2, _EC, D_NODE), *weights)
        outs.append(out_c)
    return jnp.concatenate(outs, axis=0).reshape(E)


# speedup vs baseline: 1.1461x; 1.0258x over previous
import jax
import jax.numpy as jnp
from jax.experimental import pallas as pl
from jax.experimental.pallas import tpu as pltpu
from jax.experimental.pallas import tpu_sc as plsc

N = 10000
E = 320000
D_NODE = 128
D_EATTR = 16
D_EH = 64
D_H = 128

_CHUNKS = 1
_EC = E // _CHUNKS
_BLK = 3200
_CBLOCKS = _EC // _BLK
_GW = 256


def _silu(x):
    return 0.5 * x * (jnp.tanh(0.5 * x) + 1.0)


def _sigmoid(x):
    return 0.5 * (jnp.tanh(0.5 * x) + 1.0)


def _bdot(a, b):
    return jnp.dot(a.astype(jnp.bfloat16), b.astype(jnp.bfloat16),
                   preferred_element_type=jnp.float32)


def _ln(x, g, b):
    m = jnp.mean(x, axis=-1, keepdims=True)
    v = jnp.mean((x - m) ** 2, axis=-1, keepdims=True)
    return (x - m) * jax.lax.rsqrt(v + 1e-5) * g + b


def _table_body(ne_ref, lg_ref, wsrc_ref, wdst_ref, was_ref, wad_ref, out_ref):
    ne = ne_ref[...]
    ap = jax.nn.sigmoid(lg_ref[...])
    p = jnp.dot(ne, wsrc_ref[...]) + ap * was_ref[...]
    q = jnp.dot(ne, wdst_ref[...]) + ap * wad_ref[...]
    out_ref[0] = p
    out_ref[1] = q


def _build_table(node_embedding, logits, wsrc, wdst, w_as, w_ad):
    out = pl.pallas_call(
        _table_body,
        out_shape=jax.ShapeDtypeStruct((2, N, D_NODE), jnp.float32),
    )(node_embedding, logits.reshape(N, 1), wsrc, wdst, w_as, w_ad)
    return out.reshape(2 * N, D_NODE)


def _sc_gather(table, indices):
    n_idx = indices.shape[0]
    idx2 = indices.reshape(1, n_idx)
    mesh = plsc.VectorSubcoreMesh(core_axis_name="core", subcore_axis_name="subcore")

    @pl.kernel(
        out_type=jax.ShapeDtypeStruct((n_idx, D_NODE), jnp.float32),
        mesh=mesh,
    )
    def gather_kernel(tbl_hbm, i_hbm, o_hbm):
        def body(i_vmem, o_vmem):
            pltpu.sync_copy(tbl_hbm.at[i_vmem.at[0]], o_vmem)

        pltpu.emit_pipeline(
            body,
            grid=(n_idx // _GW,),
            in_specs=[pl.BlockSpec((1, _GW), lambda i: (0, i))],
            out_specs=[pl.BlockSpec((_GW, D_NODE), lambda i: (i, 0))],
            core_axis_name=("core", "subcore"),
            dimension_semantics=(pltpu.PARALLEL,),
        )(i_hbm, o_hbm)

    return gather_kernel(table, idx2)


def _edge_body(ea_ref, dn_ref, g_ref,
               be1w_ref, be1b_ref, be1g_ref, be1be_ref,
               be2w_ref, be2b_ref, be2g_ref, be2be_ref,
               be3w_ref, be3b_ref, be3g_ref, be3be_ref,
               dg1w_ref, dg1b_ref, dg2w_ref, dg2b_ref, dgow_ref, dgob_ref,
               we_ref, wd_ref, fc1b_ref,
               fc2w_ref, fc2b_ref, outw_ref, outb_ref,
               out_ref):
    ea = ea_ref[...].astype(jnp.bfloat16)
    e = _silu(_ln(_bdot(ea, be1w_ref[...]) + be1b_ref[...],
                  be1g_ref[...], be1be_ref[...]))
    e = _silu(_ln(_bdot(e, be2w_ref[...]) + be2b_ref[...],
                  be2g_ref[...], be2be_ref[...]))
    e = _silu(_ln(_bdot(e, be3w_ref[...]) + be3b_ref[...],
                  be3g_ref[...], be3be_ref[...]))

    dn = dn_ref[...].astype(jnp.bfloat16)
    h = _silu(_bdot(dn, dg1w_ref[...]) + dg1b_ref[...])
    h = _silu(_bdot(h, dg2w_ref[...]) + dg2b_ref[...])
    dlog = jnp.sum(h * dgow_ref[...], axis=-1, keepdims=True) + dgob_ref[...]
    dp = _sigmoid(dlog)

    x1 = _silu(g_ref[0] + g_ref[1]
               + _bdot(e, we_ref[...])
               + dp * wd_ref[...] + fc1b_ref[...])
    x2 = _silu(_bdot(x1, fc2w_ref[...]) + fc2b_ref[...])
    out_ref[...] = jnp.sum(x2 * outw_ref[...], axis=-1, keepdims=True) + outb_ref[0, 0]


def _full(arr):
    nd = arr.ndim
    return pl.BlockSpec(arr.shape, lambda i, _n=nd: (0,) * _n)


def kernel(node_embedding, edge_index, edge_attr, dual_node_emb, atom_reactivity_logits,
           be1_W, be1_b, be1_g, be1_beta,
           be2_W, be2_b, be2_g, be2_beta,
           be3_W, be3_b, be3_g, be3_beta,
           dg1_W, dg1_b, dg2_W, dg2_b, dgo_W, dgo_b,
           fc1_W, fc1_b, fc2_W, fc2_b, out_W, out_b):
    src = edge_index[0].astype(jnp.int32)
    dst = edge_index[1].astype(jnp.int32)

    wsrc = fc1_W[:D_NODE]
    wdst = fc1_W[D_NODE:2 * D_NODE]
    w_e = fc1_W[2 * D_NODE:2 * D_NODE + D_EH]
    w_d = fc1_W[2 * D_NODE + D_EH:2 * D_NODE + D_EH + 1]
    w_as = fc1_W[2 * D_NODE + D_EH + 1:2 * D_NODE + D_EH + 2]
    w_ad = fc1_W[2 * D_NODE + D_EH + 2:2 * D_NODE + D_EH + 3]

    table = _build_table(node_embedding, atom_reactivity_logits, wsrc, wdst, w_as, w_ad)

    weights = [be1_W, be1_b.reshape(1, -1), be1_g.reshape(1, -1), be1_beta.reshape(1, -1),
               be2_W, be2_b.reshape(1, -1), be2_g.reshape(1, -1), be2_beta.reshape(1, -1),
               be3_W, be3_b.reshape(1, -1), be3_g.reshape(1, -1), be3_beta.reshape(1, -1),
               dg1_W, dg1_b.reshape(1, -1), dg2_W, dg2_b.reshape(1, -1),
               dgo_W.reshape(1, -1), dgo_b.reshape(1, 1),
               w_e, w_d, fc1_b.reshape(1, -1),
               fc2_W, fc2_b.reshape(1, -1), out_W.reshape(1, -1), out_b.reshape(1, 1)]

    gathers = []
    for c in range(_CHUNKS):
        s = c * _EC
        idx_c = jnp.concatenate([src[s:s + _EC], dst[s:s + _EC] + N])
        gathers.append(_sc_gather(table, idx_c))

    outs = []
    for c in range(_CHUNKS):
        in_specs = [
            pl.BlockSpec((_BLK, D_EATTR), lambda i, _c=c: (i + _c * _CBLOCKS, 0)),
            pl.BlockSpec((_BLK, D_EH), lambda i, _c=c: (i + _c * _CBLOCKS, 0)),
            pl.BlockSpec((2, _BLK, D_NODE), lambda i: (0, i, 0)),
        ] + [_full(w) for w in weights]

        out_c = pl.pallas_call(
            _edge_body,
            grid=(_CBLOCKS,),
            in_specs=in_specs,
            out_specs=pl.BlockSpec((_BLK, 1), lambda i: (i, 0)),
            out_shape=jax.ShapeDtypeStruct((_EC, 1), jnp.float32),
        )(edge_attr, dual_node_emb,
          gathers[c].reshape(2, _EC, D_NODE), *weights)
        outs.append(out_c)
    return jnp.concatenate(outs, axis=0).reshape(E)


# speedup vs baseline: 1.1640x; 1.0156x over previous
import jax
import jax.numpy as jnp
from jax.experimental import pallas as pl
from jax.experimental.pallas import tpu as pltpu
from jax.experimental.pallas import tpu_sc as plsc

N = 10000
E = 320000
D_NODE = 128
D_EATTR = 16
D_EH = 64
D_H = 128

_CHUNKS = 1
_EC = E // _CHUNKS
_BLK = 6400
_CBLOCKS = _EC // _BLK
_GW = 256


def _silu(x):
    return 0.5 * x * (jnp.tanh(0.5 * x) + 1.0)


def _sigmoid(x):
    return 0.5 * (jnp.tanh(0.5 * x) + 1.0)


def _bdot(a, b):
    return jnp.dot(a.astype(jnp.bfloat16), b.astype(jnp.bfloat16),
                   preferred_element_type=jnp.float32)


def _ln(x, g, b):
    m = jnp.mean(x, axis=-1, keepdims=True)
    v = jnp.mean((x - m) ** 2, axis=-1, keepdims=True)
    return (x - m) * jax.lax.rsqrt(v + 1e-5) * g + b


def _table_body(ne_ref, lg_ref, wsrc_ref, wdst_ref, was_ref, wad_ref, out_ref):
    ne = ne_ref[...]
    ap = jax.nn.sigmoid(lg_ref[...])
    p = jnp.dot(ne, wsrc_ref[...]) + ap * was_ref[...]
    q = jnp.dot(ne, wdst_ref[...]) + ap * wad_ref[...]
    out_ref[0] = p
    out_ref[1] = q


def _build_table(node_embedding, logits, wsrc, wdst, w_as, w_ad):
    out = pl.pallas_call(
        _table_body,
        out_shape=jax.ShapeDtypeStruct((2, N, D_NODE), jnp.float32),
    )(node_embedding, logits.reshape(N, 1), wsrc, wdst, w_as, w_ad)
    return out.reshape(2 * N, D_NODE)


def _sc_gather(table, indices):
    n_idx = indices.shape[0]
    idx2 = indices.reshape(1, n_idx)
    mesh = plsc.VectorSubcoreMesh(core_axis_name="core", subcore_axis_name="subcore")

    @pl.kernel(
        out_type=jax.ShapeDtypeStruct((n_idx, D_NODE), jnp.float32),
        mesh=mesh,
    )
    def gather_kernel(tbl_hbm, i_hbm, o_hbm):
        def body(i_vmem, o_vmem):
            pltpu.sync_copy(tbl_hbm.at[i_vmem.at[0]], o_vmem)

        pltpu.emit_pipeline(
            body,
            grid=(n_idx // _GW,),
            in_specs=[pl.BlockSpec((1, _GW), lambda i: (0, i))],
            out_specs=[pl.BlockSpec((_GW, D_NODE), lambda i: (i, 0))],
            core_axis_name=("core", "subcore"),
            dimension_semantics=(pltpu.PARALLEL,),
        )(i_hbm, o_hbm)

    return gather_kernel(table, idx2)


def _edge_body(ea_ref, dn_ref, g_ref,
               be1w_ref, be1b_ref, be1g_ref, be1be_ref,
               be2w_ref, be2b_ref, be2g_ref, be2be_ref,
               be3w_ref, be3b_ref, be3g_ref, be3be_ref,
               dg1w_ref, dg1b_ref, dg2w_ref, dg2b_ref, dgow_ref, dgob_ref,
               we_ref, wd_ref, fc1b_ref,
               fc2w_ref, fc2b_ref, outw_ref, outb_ref,
               out_ref):
    ea = ea_ref[...].astype(jnp.bfloat16)
    e = _silu(_ln(_bdot(ea, be1w_ref[...]) + be1b_ref[...],
                  be1g_ref[...], be1be_ref[...]))
    e = _silu(_ln(_bdot(e, be2w_ref[...]) + be2b_ref[...],
                  be2g_ref[...], be2be_ref[...]))
    e = _silu(_ln(_bdot(e, be3w_ref[...]) + be3b_ref[...],
                  be3g_ref[...], be3be_ref[...]))

    dn = dn_ref[...].astype(jnp.bfloat16)
    h = _silu(_bdot(dn, dg1w_ref[...]) + dg1b_ref[...])
    h = _silu(_bdot(h, dg2w_ref[...]) + dg2b_ref[...])
    dlog = jnp.sum(h * dgow_ref[...], axis=-1, keepdims=True) + dgob_ref[...]
    dp = _sigmoid(dlog)

    x1 = _silu(g_ref[0] + g_ref[1]
               + _bdot(e, we_ref[...])
               + dp * wd_ref[...] + fc1b_ref[...])
    x2 = _silu(_bdot(x1, fc2w_ref[...]) + fc2b_ref[...])
    out_ref[...] = jnp.sum(x2 * outw_ref[...], axis=-1, keepdims=True) + outb_ref[0, 0]


def _full(arr):
    nd = arr.ndim
    return pl.BlockSpec(arr.shape, lambda i, _n=nd: (0,) * _n)


def kernel(node_embedding, edge_index, edge_attr, dual_node_emb, atom_reactivity_logits,
           be1_W, be1_b, be1_g, be1_beta,
           be2_W, be2_b, be2_g, be2_beta,
           be3_W, be3_b, be3_g, be3_beta,
           dg1_W, dg1_b, dg2_W, dg2_b, dgo_W, dgo_b,
           fc1_W, fc1_b, fc2_W, fc2_b, out_W, out_b):
    src = edge_index[0].astype(jnp.int32)
    dst = edge_index[1].astype(jnp.int32)

    wsrc = fc1_W[:D_NODE]
    wdst = fc1_W[D_NODE:2 * D_NODE]
    w_e = fc1_W[2 * D_NODE:2 * D_NODE + D_EH]
    w_d = fc1_W[2 * D_NODE + D_EH:2 * D_NODE + D_EH + 1]
    w_as = fc1_W[2 * D_NODE + D_EH + 1:2 * D_NODE + D_EH + 2]
    w_ad = fc1_W[2 * D_NODE + D_EH + 2:2 * D_NODE + D_EH + 3]

    table = _build_table(node_embedding, atom_reactivity_logits, wsrc, wdst, w_as, w_ad)

    weights = [be1_W, be1_b.reshape(1, -1), be1_g.reshape(1, -1), be1_beta.reshape(1, -1),
               be2_W, be2_b.reshape(1, -1), be2_g.reshape(1, -1), be2_beta.reshape(1, -1),
               be3_W, be3_b.reshape(1, -1), be3_g.reshape(1, -1), be3_beta.reshape(1, -1),
               dg1_W, dg1_b.reshape(1, -1), dg2_W, dg2_b.reshape(1, -1),
               dgo_W.reshape(1, -1), dgo_b.reshape(1, 1),
               w_e, w_d, fc1_b.reshape(1, -1),
               fc2_W, fc2_b.reshape(1, -1), out_W.reshape(1, -1), out_b.reshape(1, 1)]

    gathers = []
    for c in range(_CHUNKS):
        s = c * _EC
        idx_c = jnp.concatenate([src[s:s + _EC], dst[s:s + _EC] + N])
        gathers.append(_sc_gather(table, idx_c))

    outs = []
    for c in range(_CHUNKS):
        in_specs = [
            pl.BlockSpec((_BLK, D_EATTR), lambda i, _c=c: (i + _c * _CBLOCKS, 0)),
            pl.BlockSpec((_BLK, D_EH), lambda i, _c=c: (i + _c * _CBLOCKS, 0)),
            pl.BlockSpec((2, _BLK, D_NODE), lambda i: (0, i, 0)),
        ] + [_full(w) for w in weights]

        out_c = pl.pallas_call(
            _edge_body,
            grid=(_CBLOCKS,),
            in_specs=in_specs,
            out_specs=pl.BlockSpec((_BLK, 1), lambda i: (i, 0)),
            out_shape=jax.ShapeDtypeStruct((_EC, 1), jnp.float32),
        )(edge_attr, dual_node_emb,
          gathers[c].reshape(2, _EC, D_NODE), *weights)
        outs.append(out_c)
    return jnp.concatenate(outs, axis=0).reshape(E)


# speedup vs baseline: 1.2089x; 1.0386x over previous
import jax
import jax.numpy as jnp
from jax.experimental import pallas as pl
from jax.experimental.pallas import tpu as pltpu
from jax.experimental.pallas import tpu_sc as plsc

N = 10000
E = 320000
D_NODE = 128
D_EATTR = 16
D_EH = 64
D_H = 128

_CHUNKS = 1
_EC = E // _CHUNKS
_BLK = 6400
_CBLOCKS = _EC // _BLK
_GW = 256


def _silu_h(y):
    return y * jnp.tanh(y) + y


def _bdot(a, b):
    return jnp.dot(a.astype(jnp.bfloat16), b.astype(jnp.bfloat16),
                   preferred_element_type=jnp.float32)


def _ln(x, g, b):
    m = jnp.mean(x, axis=-1, keepdims=True)
    v = jnp.mean((x - m) ** 2, axis=-1, keepdims=True)
    return (x - m) * jax.lax.rsqrt(v + 1e-5) * g + b


def _table_body(ne_ref, lg_ref, wsrc_ref, wdst_ref, was_ref, wad_ref, out_ref):
    ne = ne_ref[...]
    ap = jax.nn.sigmoid(lg_ref[...])
    p = jnp.dot(ne, wsrc_ref[...]) + ap * was_ref[...]
    q = jnp.dot(ne, wdst_ref[...]) + ap * wad_ref[...]
    out_ref[0] = p
    out_ref[1] = q


def _build_table(node_embedding, logits, wsrc, wdst, w_as, w_ad):
    out = pl.pallas_call(
        _table_body,
        out_shape=jax.ShapeDtypeStruct((2, N, D_NODE), jnp.float32),
    )(node_embedding, logits.reshape(N, 1), wsrc, wdst, w_as, w_ad)
    return out.reshape(2 * N, D_NODE)


def _sc_gather(table, indices):
    n_idx = indices.shape[0]
    idx2 = indices.reshape(1, n_idx)
    mesh = plsc.VectorSubcoreMesh(core_axis_name="core", subcore_axis_name="subcore")

    @pl.kernel(
        out_type=jax.ShapeDtypeStruct((n_idx, D_NODE), jnp.float32),
        mesh=mesh,
    )
    def gather_kernel(tbl_hbm, i_hbm, o_hbm):
        def body(i_vmem, o_vmem):
            pltpu.sync_copy(tbl_hbm.at[i_vmem.at[0]], o_vmem)

        pltpu.emit_pipeline(
            body,
            grid=(n_idx // _GW,),
            in_specs=[pl.BlockSpec((1, _GW), lambda i: (0, i))],
            out_specs=[pl.BlockSpec((_GW, D_NODE), lambda i: (i, 0))],
            core_axis_name=("core", "subcore"),
            dimension_semantics=(pltpu.PARALLEL,),
        )(i_hbm, o_hbm)

    return gather_kernel(table, idx2)


def _edge_body(ea_ref, dn_ref, g_ref,
               be1w_ref, be1b_ref, be1g_ref, be1be_ref,
               be2w_ref, be2b_ref, be2g_ref, be2be_ref,
               be3w_ref, be3b_ref, be3g_ref, be3be_ref,
               dg1w_ref, dg1b_ref, dg2w_ref, dg2b_ref, dgow_ref, dgob_ref,
               we_ref, wd_ref, fc1b_ref,
               fc2w_ref, fc2b_ref, outw_ref, outb_ref,
               out_ref):
    ea = ea_ref[...].astype(jnp.bfloat16)
    e = _silu_h(_ln(_bdot(ea, be1w_ref[...]) + be1b_ref[...],
                    be1g_ref[...], be1be_ref[...]))
    e = _silu_h(_ln(_bdot(e, be2w_ref[...]) + be2b_ref[...],
                    be2g_ref[...], be2be_ref[...]))
    e = _silu_h(_ln(_bdot(e, be3w_ref[...]) + be3b_ref[...],
                    be3g_ref[...], be3be_ref[...]))

    dn = dn_ref[...].astype(jnp.bfloat16)
    h = _silu_h(_bdot(dn, dg1w_ref[...]) + dg1b_ref[...])
    h = _silu_h(_bdot(h, dg2w_ref[...]) + dg2b_ref[...])
    t = jnp.sum(h * dgow_ref[...], axis=-1, keepdims=True) + dgob_ref[...]
    th = jnp.tanh(t)

    x1 = _silu_h(g_ref[0] + g_ref[1]
                 + _bdot(e, we_ref[...])
                 + th * wd_ref[...] + fc1b_ref[...])
    x2 = _silu_h(_bdot(x1, fc2w_ref[...]) + fc2b_ref[...])
    out_ref[...] = jnp.sum(x2 * outw_ref[...], axis=-1, keepdims=True) + outb_ref[0, 0]


def _full(arr):
    nd = arr.ndim
    return pl.BlockSpec(arr.shape, lambda i, _n=nd: (0,) * _n)


def kernel(node_embedding, edge_index, edge_attr, dual_node_emb, atom_reactivity_logits,
           be1_W, be1_b, be1_g, be1_beta,
           be2_W, be2_b, be2_g, be2_beta,
           be3_W, be3_b, be3_g, be3_beta,
           dg1_W, dg1_b, dg2_W, dg2_b, dgo_W, dgo_b,
           fc1_W, fc1_b, fc2_W, fc2_b, out_W, out_b):
    src = edge_index[0].astype(jnp.int32)
    dst = edge_index[1].astype(jnp.int32)

    wsrc = fc1_W[:D_NODE]
    wdst = fc1_W[D_NODE:2 * D_NODE]
    w_e = fc1_W[2 * D_NODE:2 * D_NODE + D_EH]
    w_d = fc1_W[2 * D_NODE + D_EH:2 * D_NODE + D_EH + 1]
    w_as = fc1_W[2 * D_NODE + D_EH + 1:2 * D_NODE + D_EH + 2]
    w_ad = fc1_W[2 * D_NODE + D_EH + 2:2 * D_NODE + D_EH + 3]

    table = _build_table(node_embedding, atom_reactivity_logits,
                         0.5 * wsrc, 0.5 * wdst, 0.5 * w_as, 0.5 * w_ad)

    weights = [be1_W, be1_b.reshape(1, -1),
               0.5 * be1_g.reshape(1, -1), 0.5 * be1_beta.reshape(1, -1),
               be2_W, be2_b.reshape(1, -1),
               0.5 * be2_g.reshape(1, -1), 0.5 * be2_beta.reshape(1, -1),
               be3_W, be3_b.reshape(1, -1),
               0.5 * be3_g.reshape(1, -1), 0.5 * be3_beta.reshape(1, -1),
               0.5 * dg1_W, 0.5 * dg1_b.reshape(1, -1),
               0.5 * dg2_W, 0.5 * dg2_b.reshape(1, -1),
               0.5 * dgo_W.reshape(1, -1), 0.5 * dgo_b.reshape(1, 1),
               0.5 * w_e, 0.25 * w_d,
               (0.5 * fc1_b + 0.25 * w_d.reshape(-1)).reshape(1, -1),
               0.5 * fc2_W, 0.5 * fc2_b.reshape(1, -1),
               out_W.reshape(1, -1), out_b.reshape(1, 1)]

    gathers = []
    for c in range(_CHUNKS):
        s = c * _EC
        idx_c = jnp.concatenate([src[s:s + _EC], dst[s:s + _EC] + N])
        gathers.append(_sc_gather(table, idx_c))

    outs = []
    for c in range(_CHUNKS):
        in_specs = [
            pl.BlockSpec((_BLK, D_EATTR), lambda i, _c=c: (i + _c * _CBLOCKS, 0)),
            pl.BlockSpec((_BLK, D_EH), lambda i, _c=c: (i + _c * _CBLOCKS, 0)),
            pl.BlockSpec((2, _BLK, D_NODE), lambda i: (0, i, 0)),
        ] + [_full(w) for w in weights]

        out_c = pl.pallas_call(
            _edge_body,
            grid=(_CBLOCKS,),
            in_specs=in_specs,
            out_specs=pl.BlockSpec((_BLK, 1), lambda i: (i, 0)),
            out_shape=jax.ShapeDtypeStruct((_EC, 1), jnp.float32),
        )(edge_attr, dual_node_emb,
          gathers[c].reshape(2, _EC, D_NODE), *weights)
        outs.append(out_c)
    return jnp.concatenate(outs, axis=0).reshape(E)
